# baseline (speedup 1.0000x reference)
import functools

import jax
import jax.numpy as jnp
from jax import lax
from jax.experimental import pallas as pl
from jax.experimental.pallas import tpu as pltpu
from jax.experimental.pallas import tpu_sc as plsc

_K = 32
_RADIUS = 0.12
_N1 = 4096
_N2 = 8192
_KPAD = 128
_ROWS_TC = 256
_G = 16


def _tc_mask_kernel(p1_ref, p2t_ref, sn1_ref, wi_ref, nn_ref):
    p1 = p1_ref[...]
    p2t = p2t_ref[...]
    sn1 = sn1_ref[:, 0:1]
    sn2 = jnp.sum(p2t * p2t, axis=0, keepdims=True)
    dot = lax.dot_general(p1, p2t, (((1,), (0,)), ((), ())),
                          preferred_element_type=jnp.float32)
    d2 = sn1 + sn2 - 2.0 * dot
    d = jnp.sqrt(jnp.maximum(d2, 0.0))
    within = d <= _RADIUS
    wi_ref[...] = within.astype(jnp.int8)
    nn = jnp.minimum(jnp.sum(within.astype(jnp.int32), axis=1), _K)
    nn_ref[...] = jnp.broadcast_to(nn[:, None], (p1.shape[0], 8))


def _tc_mask(p1p, p2tp, sn1b):
    grid = _N1 // _ROWS_TC
    return pl.pallas_call(
        _tc_mask_kernel,
        grid=(grid,),
        in_specs=[
            pl.BlockSpec((_ROWS_TC, _KPAD), lambda t: (t, 0)),
            pl.BlockSpec((_KPAD, _N2), lambda t: (0, 0)),
            pl.BlockSpec((_ROWS_TC, 8), lambda t: (t, 0)),
        ],
        out_specs=[
            pl.BlockSpec((_ROWS_TC, _N2), lambda t: (t, 0)),
            pl.BlockSpec((_ROWS_TC, 8), lambda t: (t, 0)),
        ],
        out_shape=[
            jax.ShapeDtypeStruct((_N1, _N2), jnp.int8),
            jax.ShapeDtypeStruct((_N1, 8), jnp.int32),
        ],
    )(p1p, p2tp, sn1b)


def _sc_body(wi_hbm, p2x_hbm, p2y_hbm, p2z_hbm, map_hbm, outs_hbm,
             p2x_v, p2y_v, p2z_v, win_v, slot_v, map_v, outs_v):
    info = plsc.get_sparse_core_info()
    nc, ns = info.num_cores, info.num_subcores
    wid = lax.axis_index("s") * nc + lax.axis_index("c")
    rows_per = _N1 // (nc * ns)
    row0 = wid * rows_per

    pltpu.sync_copy(p2x_hbm, p2x_v)
    pltpu.sync_copy(p2y_hbm, p2y_v)
    pltpu.sync_copy(p2z_hbm, p2z_v)

    iota = lax.iota(jnp.int32, 16)
    nwords = _N2 // 64

    nwpr = _N2 // 4

    def group_body(g, _):
        base = row0 + g * _G
        pltpu.sync_copy(wi_hbm.at[pl.ds(base * nwpr, _G * nwpr)], win_v)

        def row_body(i, _):
            zeros16 = jnp.zeros((16,), jnp.int32)
            slot_v[pl.ds(0, 16)] = zeros16
            slot_v[pl.ds(16, 16)] = zeros16

            def scan_cond(carry):
                w, fv = carry
                return (w < nwords) & (fv < _K)

            def scan_body(carry):
                w, fv = carry
                words = win_v[pl.ds(i * nwpr + w * 16, 16)]
                cnt = (words * 0x01010101) >> 24
                total = jnp.sum(cnt)

                csum = plsc.cumsum(cnt)
                base = fv + (csum - cnt)
                prefix = jnp.zeros((16,), jnp.int32)
                for b in range(4):
                    mb = (words >> (8 * b)) & 1
                    slot = base + prefix
                    keep = (mb == 1) & (slot < _K)
                    cols = w * 64 + 4 * iota + b
                    plsc.store_scatter(slot_v, [slot], cols, mask=keep)
                    prefix = prefix + mb
                return (w + 1, fv + total)

            _, found = lax.while_loop(scan_cond, scan_body, (0, 0))
            found = jnp.minimum(found, _K)

            m0 = slot_v[pl.ds(0, 16)]
            m1 = slot_v[pl.ds(16, 16)]
            map_v[pl.ds(i * _K, 16)] = m0
            map_v[pl.ds(i * _K + 16, 16)] = m1
            for h, mh in ((0, m0), (1, m1)):
                valid = (iota + 16 * h) < found
                gx = jnp.where(valid, plsc.load_gather(p2x_v, [mh]), 0.0)
                gy = jnp.where(valid, plsc.load_gather(p2y_v, [mh]), 0.0)
                gz = jnp.where(valid, plsc.load_gather(p2z_v, [mh]), 0.0)
                cbase = i * (3 * _K) + (iota + 16 * h) * 3
                plsc.store_scatter(outs_v, [cbase], gx)
                plsc.store_scatter(outs_v, [cbase + 1], gy)
                plsc.store_scatter(outs_v, [cbase + 2], gz)
            return 0

        lax.fori_loop(0, _G, row_body, 0)
        pltpu.sync_copy(map_v, map_hbm.at[pl.ds(base * _K, _G * _K)])
        pltpu.sync_copy(outs_v, outs_hbm.at[pl.ds(base * 3 * _K, _G * 3 * _K)])
        return 0

    lax.fori_loop(0, rows_per // _G, group_body, 0)


@functools.partial(
    pl.kernel,
    mesh=plsc.VectorSubcoreMesh(core_axis_name="c", subcore_axis_name="s"),
    compiler_params=pltpu.CompilerParams(needs_layout_passes=False),
    out_type=[
        jax.ShapeDtypeStruct((_N1 * _K,), jnp.int32),
        jax.ShapeDtypeStruct((_N1 * 3 * _K,), jnp.float32),
    ],
    scratch_types=[
        pltpu.VMEM((_N2,), jnp.float32),
        pltpu.VMEM((_N2,), jnp.float32),
        pltpu.VMEM((_N2,), jnp.float32),
        pltpu.VMEM((_G * (_N2 // 4),), jnp.int32),
        pltpu.VMEM((32,), jnp.int32),
        pltpu.VMEM((_G * _K,), jnp.int32),
        pltpu.VMEM((_G * 3 * _K,), jnp.float32),
    ],
)
def _sc_extract(*args):
    _sc_body(*args)


def kernel(points1, points2):
    p1 = points1[0]
    p2 = points2[0]
    p1p = jnp.pad(p1, ((0, 0), (0, _KPAD - 3)))
    p2tp = jnp.pad(p2.T, ((0, _KPAD - 3), (0, 0)))
    sn1 = jnp.sum(p1 * p1, axis=1)
    sn1b = jnp.broadcast_to(sn1[:, None], (_N1, 8))
    wi8, nn8 = _tc_mask(p1p, p2tp, sn1b)
    p2x = p2[:, 0]
    p2y = p2[:, 1]
    p2z = p2[:, 2]
    wi_words = jax.lax.bitcast_convert_type(
        wi8.reshape(_N1, _N2 // 4, 4), jnp.int32).reshape(-1)
    mapping, outs = _sc_extract(wi_words, p2x, p2y, p2z)
    num_neighbors = nn8[:, 0]
    mapping = mapping.reshape(_N1, _K)
    outputs = outs.reshape(_N1, _K, 3)
    return mapping[None], num_neighbors[None], outputs[None]


# speedup vs baseline: 1.1211x; 1.1211x over previous
import functools

import jax
import jax.numpy as jnp
from jax import lax
from jax.experimental import pallas as pl
from jax.experimental.pallas import tpu as pltpu
from jax.experimental.pallas import tpu_sc as plsc

_K = 32
_RADIUS = 0.12
_N1 = 4096
_N2 = 8192
_KPAD = 128
_ROWS_TC = 256
_NCH = 128
_G = 16


def _tc_mask_kernel(p1_ref, p2t_ref, sn1_ref, blk_ref, tri_ref,
                    wi_ref, nn_ref, ids_ref, bases_ref):
    p1 = p1_ref[...]
    p2t = p2t_ref[...]
    sn1 = sn1_ref[:, 0:1]
    sn2 = jnp.sum(p2t * p2t, axis=0, keepdims=True)
    dot = lax.dot_general(p1, p2t, (((1,), (0,)), ((), ())),
                          preferred_element_type=jnp.float32)
    d2 = sn1 + sn2 - 2.0 * dot
    d = jnp.sqrt(jnp.maximum(d2, 0.0))
    within = d <= _RADIUS
    wi_ref[...] = within.astype(jnp.int8)
    nn = jnp.minimum(jnp.sum(within.astype(jnp.int32), axis=1), _K)
    nn_ref[...] = jnp.broadcast_to(nn[:, None], (p1.shape[0], 8))

    wf = within.astype(jnp.float32)
    ccnt = lax.dot_general(wf, blk_ref[...], (((1,), (0,)), ((), ())),
                           preferred_element_type=jnp.float32)
    excl = lax.dot_general(ccnt, tri_ref[...], (((1,), (0,)), ((), ())),
                           preferred_element_type=jnp.float32)
    listed = (ccnt > 0.0) & (excl < float(_K))
    lexcl = lax.dot_general(listed.astype(jnp.float32), tri_ref[...],
                            (((1,), (0,)), ((), ())),
                            preferred_element_type=jnp.float32)
    cvec = jax.lax.broadcasted_iota(jnp.int32, (p1.shape[0], _NCH), 1).astype(jnp.float32)
    jot = jax.lax.broadcasted_iota(jnp.int32, (p1.shape[0], _K), 1).astype(jnp.float32)
    ids = jnp.zeros((p1.shape[0], _K), jnp.float32)
    bases = jnp.zeros((p1.shape[0], _K), jnp.float32)
    for j in range(_K):
        sel = listed & (lexcl == float(j))
        idj = jnp.sum(jnp.where(sel, cvec, 0.0), axis=1, keepdims=True)
        bj = jnp.sum(jnp.where(sel, excl, 0.0), axis=1, keepdims=True)
        onehot = (jot == float(j)).astype(jnp.float32)
        ids = ids + idj * onehot
        bases = bases + bj * onehot
    ids_ref[...] = ids.astype(jnp.int32)
    bases_ref[...] = bases.astype(jnp.int32)


def _tc_mask(p1p, p2tp, sn1b, blk, tri):
    grid = _N1 // _ROWS_TC
    return pl.pallas_call(
        _tc_mask_kernel,
        grid=(grid,),
        in_specs=[
            pl.BlockSpec((_ROWS_TC, _KPAD), lambda t: (t, 0)),
            pl.BlockSpec((_KPAD, _N2), lambda t: (0, 0)),
            pl.BlockSpec((_ROWS_TC, 8), lambda t: (t, 0)),
            pl.BlockSpec((_N2, _NCH), lambda t: (0, 0)),
            pl.BlockSpec((_NCH, _NCH), lambda t: (0, 0)),
        ],
        out_specs=[
            pl.BlockSpec((_ROWS_TC, _N2), lambda t: (t, 0)),
            pl.BlockSpec((_ROWS_TC, 8), lambda t: (t, 0)),
            pl.BlockSpec((_ROWS_TC, _K), lambda t: (t, 0)),
            pl.BlockSpec((_ROWS_TC, _K), lambda t: (t, 0)),
        ],
        out_shape=[
            jax.ShapeDtypeStruct((_N1, _N2), jnp.int8),
            jax.ShapeDtypeStruct((_N1, 8), jnp.int32),
            jax.ShapeDtypeStruct((_N1, _K), jnp.int32),
            jax.ShapeDtypeStruct((_N1, _K), jnp.int32),
        ],
    )(p1p, p2tp, sn1b, blk, tri)


def _sc_body(wi_hbm, ids_hbm, bases_hbm, nn_hbm, p2x_hbm, p2y_hbm, p2z_hbm,
             map_hbm, outs_hbm,
             p2x_v, p2y_v, p2z_v, win_v, ids_v, bases_v, nn_v,
             slot_v, map_v, outs_v):
    info = plsc.get_sparse_core_info()
    nc, ns = info.num_cores, info.num_subcores
    wid = lax.axis_index("s") * nc + lax.axis_index("c")
    rows_per = _N1 // (nc * ns)
    row0 = wid * rows_per

    pltpu.sync_copy(p2x_hbm, p2x_v)
    pltpu.sync_copy(p2y_hbm, p2y_v)
    pltpu.sync_copy(p2z_hbm, p2z_v)

    iota = lax.iota(jnp.int32, 16)
    nwpr = _N2 // 4

    def group_body(g, _):
        base = row0 + g * _G
        pltpu.sync_copy(wi_hbm.at[pl.ds(base * nwpr, _G * nwpr)], win_v)
        pltpu.sync_copy(ids_hbm.at[pl.ds(base * _K, _G * _K)], ids_v)
        pltpu.sync_copy(bases_hbm.at[pl.ds(base * _K, _G * _K)], bases_v)
        pltpu.sync_copy(nn_hbm.at[pl.ds(base * 8, _G * 8)], nn_v)

        def row_body(i, _):
            zeros16 = jnp.zeros((16,), jnp.int32)
            slot_v[pl.ds(0, 16)] = zeros16
            slot_v[pl.ds(16, 16)] = zeros16

            def chunk_body(j, _):
                lid = jnp.full((16,), i * _K + j, jnp.int32)
                idspl = plsc.load_gather(ids_v, [lid])
                bspl = plsc.load_gather(bases_v, [lid])
                words = plsc.load_gather(
                    win_v, [i * nwpr + idspl * 16 + iota])
                cnt = (words * 0x01010101) >> 24
                csum = plsc.cumsum(cnt)
                wbase = bspl + (csum - cnt)
                prefix = jnp.zeros((16,), jnp.int32)
                for b in range(4):
                    mb = (words >> (8 * b)) & 1
                    slot = wbase + prefix
                    keep = (mb == 1) & (slot < _K)
                    cols = idspl * 64 + 4 * iota + b
                    plsc.store_scatter(slot_v, [slot], cols, mask=keep)
                    prefix = prefix + mb
                return 0

            lax.fori_loop(0, _K, chunk_body, 0)

            found = plsc.load_gather(nn_v, [jnp.full((16,), i * 8, jnp.int32)])
            m0 = slot_v[pl.ds(0, 16)]
            m1 = slot_v[pl.ds(16, 16)]
            map_v[pl.ds(i * _K, 16)] = m0
            map_v[pl.ds(i * _K + 16, 16)] = m1
            for h, mh in ((0, m0), (1, m1)):
                valid = (iota + 16 * h) < found
                gx = jnp.where(valid, plsc.load_gather(p2x_v, [mh]), 0.0)
                gy = jnp.where(valid, plsc.load_gather(p2y_v, [mh]), 0.0)
                gz = jnp.where(valid, plsc.load_gather(p2z_v, [mh]), 0.0)
                cbase = i * (3 * _K) + (iota + 16 * h) * 3
                plsc.store_scatter(outs_v, [cbase], gx)
                plsc.store_scatter(outs_v, [cbase + 1], gy)
                plsc.store_scatter(outs_v, [cbase + 2], gz)
            return 0

        lax.fori_loop(0, _G, row_body, 0)
        pltpu.sync_copy(map_v, map_hbm.at[pl.ds(base * _K, _G * _K)])
        pltpu.sync_copy(outs_v, outs_hbm.at[pl.ds(base * 3 * _K, _G * 3 * _K)])
        return 0

    lax.fori_loop(0, rows_per // _G, group_body, 0)


@functools.partial(
    pl.kernel,
    mesh=plsc.VectorSubcoreMesh(core_axis_name="c", subcore_axis_name="s"),
    compiler_params=pltpu.CompilerParams(needs_layout_passes=False),
    out_type=[
        jax.ShapeDtypeStruct((_N1 * _K,), jnp.int32),
        jax.ShapeDtypeStruct((_N1 * 3 * _K,), jnp.float32),
    ],
    scratch_types=[
        pltpu.VMEM((_N2,), jnp.float32),
        pltpu.VMEM((_N2,), jnp.float32),
        pltpu.VMEM((_N2,), jnp.float32),
        pltpu.VMEM((_G * (_N2 // 4),), jnp.int32),
        pltpu.VMEM((_G * _K,), jnp.int32),
        pltpu.VMEM((_G * _K,), jnp.int32),
        pltpu.VMEM((_G * 8,), jnp.int32),
        pltpu.VMEM((32,), jnp.int32),
        pltpu.VMEM((_G * _K,), jnp.int32),
        pltpu.VMEM((_G * 3 * _K,), jnp.float32),
    ],
)
def _sc_extract(*args):
    _sc_body(*args)


def kernel(points1, points2):
    p1 = points1[0]
    p2 = points2[0]
    p1p = jnp.pad(p1, ((0, 0), (0, _KPAD - 3)))
    p2tp = jnp.pad(p2.T, ((0, _KPAD - 3), (0, 0)))
    sn1 = jnp.sum(p1 * p1, axis=1)
    sn1b = jnp.broadcast_to(sn1[:, None], (_N1, 8))
    colg = jnp.arange(_N2, dtype=jnp.int32) // 64
    blk = (colg[:, None] == jnp.arange(_NCH, dtype=jnp.int32)[None, :])
    blk = blk.astype(jnp.float32)
    a = jnp.arange(_NCH, dtype=jnp.int32)
    tri = (a[:, None] < a[None, :]).astype(jnp.float32)
    wi8, nn8, ids, bases = _tc_mask(p1p, p2tp, sn1b, blk, tri)
    wi_words = jax.lax.bitcast_convert_type(
        wi8.reshape(_N1, _N2 // 4, 4), jnp.int32).reshape(-1)
    p2x = p2[:, 0]
    p2y = p2[:, 1]
    p2z = p2[:, 2]
    mapping, outs = _sc_extract(wi_words, ids.reshape(-1), bases.reshape(-1),
                                nn8.reshape(-1), p2x, p2y, p2z)
    num_neighbors = nn8[:, 0]
    mapping = mapping.reshape(_N1, _K)
    outputs = outs.reshape(_N1, _K, 3)
    return mapping[None], num_neighbors[None], outputs[None]
